# baseline (speedup 1.0000x reference)
import jax
import jax.numpy as jnp
from jax import lax
from jax.experimental import pallas as pl
from jax.experimental.pallas import tpu as pltpu
from jax.experimental.pallas import tpu_sc as plsc

F = 26
V = 100000
D = 32
B = 16384
L = 16
NC, NS = 2, 16
NW = NC * NS
BPW = B // NW
CHUNK = 128
CPF = BPW // CHUNK
NCHUNK = CPF * F
XPW = BPW * F

TW = 512
TG = (V + TW - 1) // TW
V2 = TG * TW


def _tc_body(tt_ref, out_ref):
    y = tt_ref[0].T
    out_ref[0] = jnp.concatenate(
        [y[0:128], y[128:256], y[256:384], y[384:512]], axis=1)


@jax.jit
def _relayout(tt):
    return pl.pallas_call(
        _tc_body,
        grid=(F, TG),
        in_specs=[pl.BlockSpec((1, D, TW), lambda f, k: (f, 0, k))],
        out_specs=pl.BlockSpec((1, TW // 4, 128), lambda f, k: (f, k, 0)),
        out_shape=jax.ShapeDtypeStruct((F, V2 * D // 128, 128), jnp.float32),
    )(tt)


def _body(xt_hbm, tbl_hbm, out_hbm, xv, idxq, bufa, bufb, acc, sem):
    wid = lax.axis_index("s") * NC + lax.axis_index("c")

    pltpu.sync_copy(xt_hbm.at[pl.ds(wid * XPW, XPW)], xv)

    def ib(j, _):
        v = xv[pl.ds(j * L, L)]
        rowp = (((v >> 9) << 7) + (v & 127)) * 4 + ((v >> 7) & 3)
        idxq[j // 8, pl.ds((j % 8) * L, L)] = rowp
        return 0
    lax.fori_loop(0, XPW // L, ib, 0)

    zv = jnp.zeros((L,), jnp.float32)
    def zb(r, _):
        acc[r, pl.ds(0, L)] = zv
        acc[r, pl.ds(L, L)] = zv
        return 0
    lax.fori_loop(0, BPW, zb, 0)

    def fire(c, dst):
        f = c // CPF
        pltpu.async_copy(tbl_hbm.at[f].at[idxq.at[c]], dst, sem)

    def drain(c, src):
        f = c // CPF
        pltpu.make_async_copy(tbl_hbm.at[f].at[idxq.at[c]], src, sem).wait()

    def accum(c, src):
        r0 = (c % CPF) * CHUNK
        def rb(r8, _):
            for rr in range(8):
                r = r8 * 8 + rr
                plsc.addupdate(acc.at[r0 + r, pl.ds(0, L)], src[r, pl.ds(0, L)])
                plsc.addupdate(acc.at[r0 + r, pl.ds(L, L)], src[r, pl.ds(L, L)])
            return 0
        lax.fori_loop(0, CHUNK // 8, rb, 0)

    fire(0, bufa)
    def pb(p, _):
        e = 2 * p
        fire(e + 1, bufb)
        drain(e, bufa)
        accum(e, bufa)
        @pl.when(p < NCHUNK // 2 - 1)
        def _():
            fire(e + 2, bufa)
        drain(e + 1, bufb)
        accum(e + 1, bufb)
        return 0
    lax.fori_loop(0, NCHUNK // 2, pb, 0)

    pltpu.sync_copy(acc, out_hbm.at[pl.ds(wid * BPW, BPW)])


@jax.jit
def _run(xt, tbl):
    mesh = plsc.VectorSubcoreMesh(core_axis_name="c", subcore_axis_name="s")
    return pl.kernel(
        _body,
        out_type=jax.ShapeDtypeStruct((B, D), jnp.float32),
        mesh=mesh,
        scratch_types=[
            pltpu.VMEM((XPW,), jnp.int32),
            pltpu.VMEM((NCHUNK, CHUNK), jnp.int32),
            pltpu.VMEM((CHUNK, D), jnp.float32),
            pltpu.VMEM((CHUNK, D), jnp.float32),
            pltpu.VMEM((BPW, D), jnp.float32),
            pltpu.SemaphoreType.DMA,
        ],
        compiler_params=pltpu.CompilerParams(use_tc_tiling_on_sc=False),
    )(xt, tbl)


def kernel(x, tables):
    xt = x.reshape(NW, BPW, F).transpose(0, 2, 1).reshape(B * F)
    tt = tables.transpose(0, 2, 1)
    tbl = _relayout(tt).reshape(F, V2, D)
    return _run(xt, tbl)


# speedup vs baseline: 1.4028x; 1.4028x over previous
import numpy as np
import jax
import jax.numpy as jnp
from jax import lax
from jax.experimental import pallas as pl
from jax.experimental.pallas import tpu as pltpu
from jax.experimental.pallas import tpu_sc as plsc

F = 26
V = 100000
D = 32
B = 16384
L = 16
NC, NS = 2, 16
NW = NC * NS
BPW = B // NW
CHUNK = 128
CPF = BPW // CHUNK
NCHUNK = CPF * F
XPW = BPW * F

TW = 512
TG = (V + TW - 1) // TW
V2 = TG * TW
KT = 10

_TQ = np.zeros((4, D, 128), np.float32)
for _q in range(4):
    for _j in range(D):
        _TQ[_q, _j, 32 * _q + _j] = 1.0


def _tc_body(tq_ref, tt_ref, out_ref):
    a = tt_ref[0]
    acc = None
    for q in range(4):
        u = lax.dot_general(a, tq_ref[q], (((0,), (0,)), ((), ())),
                            preferred_element_type=jnp.float32)
        w = u[128 * q:128 * (q + 1)]
        acc = w if acc is None else acc + w
    out_ref[0] = acc


@jax.jit
def _relayout(tq, tt):
    return pl.pallas_call(
        _tc_body,
        grid=(KT, TG),
        in_specs=[
            pl.BlockSpec((4, D, 128), lambda f, k: (0, 0, 0)),
            pl.BlockSpec((1, D, TW), lambda f, k: (f, 0, k)),
        ],
        out_specs=pl.BlockSpec((1, TW // 4, 128), lambda f, k: (f, k, 0)),
        out_shape=jax.ShapeDtypeStruct((KT, V2 * D // 128, 128), jnp.float32),
    )(tq, tt)


def _body(xt_hbm, tbla_hbm, tblb_hbm, out_hbm, xv, idxq, bufa, bufb, acc, sem):
    wid = lax.axis_index("s") * NC + lax.axis_index("c")

    pltpu.sync_copy(xt_hbm.at[pl.ds(wid * XPW, XPW)], xv)

    def ib(j, _):
        v = xv[pl.ds(j * L, L)]
        rowp = (((v >> 9) << 7) + (v & 127)) * 4 + ((v >> 7) & 3)
        idxq[j // 8, pl.ds((j % 8) * L, L)] = jnp.where(j < KT * (BPW // L),
                                                        rowp, v)
        return 0
    lax.fori_loop(0, XPW // L, ib, 0)

    zv = jnp.zeros((L,), jnp.float32)
    def zb(r, _):
        acc[r, pl.ds(0, L)] = zv
        acc[r, pl.ds(L, L)] = zv
        return 0
    lax.fori_loop(0, BPW, zb, 0)

    def fire(c, dst):
        f = c // CPF
        @pl.when(f < KT)
        def _():
            pltpu.async_copy(tbla_hbm.at[f].at[idxq.at[c]], dst, sem)
        @pl.when(f >= KT)
        def _():
            pltpu.async_copy(tblb_hbm.at[f - KT].at[idxq.at[c]], dst, sem)

    def drain(c, src):
        f = c // CPF
        @pl.when(f < KT)
        def _():
            pltpu.make_async_copy(tbla_hbm.at[f].at[idxq.at[c]], src, sem).wait()
        @pl.when(f >= KT)
        def _():
            pltpu.make_async_copy(tblb_hbm.at[f - KT].at[idxq.at[c]], src,
                                  sem).wait()

    def accum(c, src):
        r0 = (c % CPF) * CHUNK
        def rb(r8, _):
            for rr in range(8):
                r = r8 * 8 + rr
                plsc.addupdate(acc.at[r0 + r, pl.ds(0, L)], src[r, pl.ds(0, L)])
                plsc.addupdate(acc.at[r0 + r, pl.ds(L, L)], src[r, pl.ds(L, L)])
            return 0
        lax.fori_loop(0, CHUNK // 8, rb, 0)

    fire(0, bufa)
    def pb(p, _):
        e = 2 * p
        fire(e + 1, bufb)
        drain(e, bufa)
        accum(e, bufa)
        @pl.when(p < NCHUNK // 2 - 1)
        def _():
            fire(e + 2, bufa)
        drain(e + 1, bufb)
        accum(e + 1, bufb)
        return 0
    lax.fori_loop(0, NCHUNK // 2, pb, 0)

    pltpu.sync_copy(acc, out_hbm.at[pl.ds(wid * BPW, BPW)])


@jax.jit
def _run(xt, tbla, tblb):
    mesh = plsc.VectorSubcoreMesh(core_axis_name="c", subcore_axis_name="s")
    return pl.kernel(
        _body,
        out_type=jax.ShapeDtypeStruct((B, D), jnp.float32),
        mesh=mesh,
        scratch_types=[
            pltpu.VMEM((XPW,), jnp.int32),
            pltpu.VMEM((NCHUNK, CHUNK), jnp.int32),
            pltpu.VMEM((CHUNK, D), jnp.float32),
            pltpu.VMEM((CHUNK, D), jnp.float32),
            pltpu.VMEM((BPW, D), jnp.float32),
            pltpu.SemaphoreType.DMA,
        ],
        compiler_params=pltpu.CompilerParams(use_tc_tiling_on_sc=False),
    )(xt, tbla, tblb)


def kernel(x, tables):
    xt = x.reshape(NW, BPW, F).transpose(0, 2, 1).reshape(B * F)
    tt = tables.transpose(0, 2, 1)[:KT]
    tbla = _relayout(jnp.asarray(_TQ), tt).reshape(KT, V2, D)
    tblb = tables[KT:]
    return _run(xt, tbla, tblb)


# speedup vs baseline: 1.5196x; 1.0832x over previous
import jax
import jax.numpy as jnp
from jax import lax
from jax.experimental import pallas as pl
from jax.experimental.pallas import tpu as pltpu
from jax.experimental.pallas import tpu_sc as plsc

F = 26
V = 100000
D = 32
B = 16384
L = 16
NC, NS = 2, 16
NW = NC * NS
BPW = B // NW
CHUNK = 128
CPB = BPW // CHUNK
NCHUNK = CPB * F
XPW = BPW * F


def _body(xt_hbm, tbl_hbm, out_hbm, idxq, ga, gb, acc, sem):
    wid = lax.axis_index("s") * NC + lax.axis_index("c")

    pltpu.sync_copy(xt_hbm.at[pl.ds(wid * NCHUNK, NCHUNK)], idxq)

    zv = jnp.zeros((L,), jnp.float32)
    def zb(z, _):
        acc[z // D, pl.ds((z % D) * L, L)] = zv
        return 0
    lax.fori_loop(0, D * (BPW // L), zb, 0)

    def fire(w, dst):
        c = w // D
        j = w % D
        pltpu.async_copy(tbl_hbm.at[c // CPB, j].at[idxq.at[c]], dst, sem)

    def drain(w, src):
        c = w // D
        j = w % D
        pltpu.make_async_copy(tbl_hbm.at[c // CPB, j].at[idxq.at[c]], src,
                              sem).wait()

    def accum(w, src):
        c = w // D
        j = w % D
        col0 = (c % CPB) * CHUNK
        for t in range(CHUNK // L):
            plsc.addupdate(acc.at[j, pl.ds(col0 + t * L, L)],
                           src[pl.ds(t * L, L)])

    NWORK = NCHUNK * D
    fire(0, ga)
    def pb(p, _):
        e = 2 * p
        fire(e + 1, gb)
        drain(e, ga)
        accum(e, ga)
        @pl.when(p < NWORK // 2 - 1)
        def _():
            fire(e + 2, ga)
        drain(e + 1, gb)
        accum(e + 1, gb)
        return 0
    lax.fori_loop(0, NWORK // 2, pb, 0)

    pltpu.sync_copy(acc, out_hbm.at[:, pl.ds(wid * BPW, BPW)])


@jax.jit
def _run(xt, tt):
    mesh = plsc.VectorSubcoreMesh(core_axis_name="c", subcore_axis_name="s")
    return pl.kernel(
        _body,
        out_type=jax.ShapeDtypeStruct((D, B), jnp.float32),
        mesh=mesh,
        scratch_types=[
            pltpu.VMEM((NCHUNK, CHUNK), jnp.int32),
            pltpu.VMEM((CHUNK,), jnp.float32),
            pltpu.VMEM((CHUNK,), jnp.float32),
            pltpu.VMEM((D, BPW), jnp.float32),
            pltpu.SemaphoreType.DMA,
        ],
        compiler_params=pltpu.CompilerParams(use_tc_tiling_on_sc=False),
    )(xt, tt)


def kernel(x, tables):
    xt = x.reshape(NW, BPW, F).transpose(0, 2, 1).reshape(-1, CHUNK)
    tt = tables.transpose(0, 2, 1)
    return _run(xt, tt).T


# speedup vs baseline: 2.5595x; 1.6843x over previous
import jax
import jax.numpy as jnp
from jax import lax
from jax.experimental import pallas as pl
from jax.experimental.pallas import tpu as pltpu
from jax.experimental.pallas import tpu_sc as plsc

F = 26
V = 100000
D = 32
B = 16384
L = 16
NC, NS = 2, 16
NW = NC * NS
BPW = B // NW
CHUNK = 128
CPF = BPW // CHUNK
NCHUNK = CPF * F


def _body(xt_hbm, tbl_hbm, out_hbm, idxq, bufa, bufb, bufc, bufd, acc, sem):
    wid = lax.axis_index("s") * NC + lax.axis_index("c")

    pltpu.sync_copy(xt_hbm.at[pl.ds(wid * NCHUNK, NCHUNK)], idxq)

    zv = jnp.zeros((L,), jnp.float32)
    def zb(r, _):
        acc[r, pl.ds(0, L)] = zv
        acc[r, pl.ds(L, L)] = zv
        return 0
    lax.fori_loop(0, BPW, zb, 0)

    def fire(c, dst):
        pltpu.async_copy(tbl_hbm.at[c // CPF].at[idxq.at[c]], dst, sem)

    def drain(c, src):
        pltpu.make_async_copy(tbl_hbm.at[c // CPF].at[idxq.at[c]], src,
                              sem).wait()

    def accum(c, src):
        r0 = (c % CPF) * CHUNK
        def rb(r8, _):
            for rr in range(8):
                r = r8 * 8 + rr
                plsc.addupdate(acc.at[r0 + r, pl.ds(0, L)], src[r, pl.ds(0, L)])
                plsc.addupdate(acc.at[r0 + r, pl.ds(L, L)], src[r, pl.ds(L, L)])
            return 0
        lax.fori_loop(0, CHUNK // 8, rb, 0)

    fire(0, bufa)
    fire(1, bufb)
    fire(2, bufc)
    def pb(p, _):
        e = 4 * p
        fire(e + 3, bufd)
        drain(e, bufa)
        accum(e, bufa)
        @pl.when(p < NCHUNK // 4 - 1)
        def _():
            fire(e + 4, bufa)
        drain(e + 1, bufb)
        accum(e + 1, bufb)
        @pl.when(p < NCHUNK // 4 - 1)
        def _():
            fire(e + 5, bufb)
        drain(e + 2, bufc)
        accum(e + 2, bufc)
        @pl.when(p < NCHUNK // 4 - 1)
        def _():
            fire(e + 6, bufc)
        drain(e + 3, bufd)
        accum(e + 3, bufd)
        return 0
    lax.fori_loop(0, NCHUNK // 4, pb, 0)

    pltpu.sync_copy(acc, out_hbm.at[pl.ds(wid * BPW, BPW)])


@jax.jit
def _run(xt, tbl):
    mesh = plsc.VectorSubcoreMesh(core_axis_name="c", subcore_axis_name="s")
    return pl.kernel(
        _body,
        out_type=jax.ShapeDtypeStruct((B, D), jnp.float32),
        mesh=mesh,
        scratch_types=[
            pltpu.VMEM((NCHUNK, CHUNK), jnp.int32),
            pltpu.VMEM((CHUNK, D), jnp.float32),
            pltpu.VMEM((CHUNK, D), jnp.float32),
            pltpu.VMEM((CHUNK, D), jnp.float32),
            pltpu.VMEM((CHUNK, D), jnp.float32),
            pltpu.VMEM((BPW, D), jnp.float32),
            pltpu.SemaphoreType.DMA,
        ],
        compiler_params=pltpu.CompilerParams(use_tc_tiling_on_sc=False),
    )(xt, tbl)


def kernel(x, tables):
    xt = x.reshape(NW, BPW, F).transpose(0, 2, 1).reshape(-1, CHUNK)
    return _run(xt, tables)
